# Initial kernel scaffold; baseline (speedup 1.0000x reference)
import functools

import jax
import jax.numpy as jnp
from jax import lax
from jax.experimental import pallas as pl
from jax.experimental.pallas import tpu as pltpu
from jax.experimental.pallas import tpu_sc as plsc

_NUM_CLASSES = 1000000
_EMBED_DIM = 32
_BATCH = 16384
_HIST = 200
_N = _BATCH * _HIST

_NC = 2
_NS = 16
_NW = _NC * _NS
_PER_W = _N // _NW
_CHUNK = 2048
_NCHUNK = _PER_W // _CHUNK


def _lookup_body(idx_hbm, table_hbm, out_hbm, idx_v, rows_v, gsem):
    wid = lax.axis_index("s") * _NC + lax.axis_index("c")
    base = wid * _PER_W

    def step(c, carry):
        off = base + c * _CHUNK
        pltpu.sync_copy(idx_hbm.at[pl.ds(off, _CHUNK)], idx_v)
        pltpu.async_copy(table_hbm.at[idx_v], rows_v, gsem).wait()
        pltpu.sync_copy(rows_v, out_hbm.at[pl.ds(off, _CHUNK)])
        return carry

    lax.fori_loop(0, _NCHUNK, step, 0)


@jax.jit
def _lookup(idx, table):
    f = pl.kernel(
        _lookup_body,
        out_type=jax.ShapeDtypeStruct((_N, _EMBED_DIM), jnp.float32),
        mesh=plsc.VectorSubcoreMesh(core_axis_name="c", subcore_axis_name="s"),
        scratch_types=[
            pltpu.VMEM((_CHUNK,), jnp.int32),
            pltpu.VMEM((_CHUNK, _EMBED_DIM), jnp.float32),
            pltpu.SemaphoreType.DMA,
        ],
    )
    return f(idx, table)


def kernel(labels, embeddings):
    idx = labels.reshape(-1).astype(jnp.int32)
    out = _lookup(idx, embeddings)
    return out.reshape(_BATCH, _HIST, _EMBED_DIM)


# speedup vs baseline: 4.9489x; 4.9489x over previous
import functools

import jax
import jax.numpy as jnp
from jax import lax
from jax.experimental import pallas as pl
from jax.experimental.pallas import tpu as pltpu
from jax.experimental.pallas import tpu_sc as plsc

_NUM_CLASSES = 1000000
_EMBED_DIM = 32
_BATCH = 16384
_HIST = 200
_N = _BATCH * _HIST

_NC = 2
_NS = 16
_NW = _NC * _NS
_PER_W = _N // _NW
_CHUNK = 2048
_NCHUNK = _PER_W // _CHUNK


def _lookup_body(idx_hbm, table_hbm, out_hbm, idx_v, rows_v, gsem):
    wid = lax.axis_index("s") * _NC + lax.axis_index("c")
    base = wid * _PER_W

    def step(c, carry):
        off = base + c * _CHUNK
        pltpu.sync_copy(idx_hbm.at[pl.ds(off, _CHUNK)], idx_v)
        pltpu.async_copy(table_hbm.at[idx_v], rows_v, gsem).wait()
        pltpu.sync_copy(rows_v, out_hbm.at[pl.ds(off, _CHUNK)])
        return carry

    lax.fori_loop(0, _NCHUNK, step, 0)


@jax.jit
def _lookup(idx, table):
    f = pl.kernel(
        _lookup_body,
        out_type=jax.ShapeDtypeStruct((_N, _EMBED_DIM), jnp.float32),
        mesh=plsc.VectorSubcoreMesh(core_axis_name="c", subcore_axis_name="s"),
        scratch_types=[
            pltpu.VMEM((_CHUNK,), jnp.int32),
            pltpu.VMEM((_CHUNK, _EMBED_DIM), jnp.float32),
            pltpu.SemaphoreType.DMA,
        ],
        compiler_params=pltpu.CompilerParams(use_tc_tiling_on_sc=False),
    )
    return f(idx, table)


def kernel(labels, embeddings):
    idx = labels.reshape(-1).astype(jnp.int32)
    out = _lookup(idx, embeddings)
    return out.reshape(_BATCH, _HIST, _EMBED_DIM)


# speedup vs baseline: 5.0352x; 1.0174x over previous
import jax
import jax.numpy as jnp
from jax import lax
from jax.experimental import pallas as pl
from jax.experimental.pallas import tpu as pltpu
from jax.experimental.pallas import tpu_sc as plsc

_NUM_CLASSES = 1000000
_EMBED_DIM = 32
_BATCH = 16384
_HIST = 200
_N = _BATCH * _HIST

_NC = 2
_NS = 16
_NW = _NC * _NS
_PER_W = _N // _NW
_CHUNK = 1600
_NCHUNK = _PER_W // _CHUNK
_NBUF = 2


def _lookup_body(idx_hbm, table_hbm, out_hbm, idx_v, rows_v, isem, gsem, osem):
    wid = lax.axis_index("s") * _NC + lax.axis_index("c")
    base = wid * _PER_W

    def idx_start(c, b):
        return pltpu.async_copy(
            idx_hbm.at[pl.ds(base + c * _CHUNK, _CHUNK)], idx_v.at[b], isem.at[b]
        )

    for b in range(_NBUF):
        idx_start(b, b)

    def idx_wait(b):
        pltpu.make_async_copy(
            idx_hbm.at[pl.ds(base, _CHUNK)], idx_v.at[b], isem.at[b]
        ).wait()

    def outer(o, carry):
        for b in range(_NBUF):
            c = o * _NBUF + b
            idx_wait(b)
            @pl.when(o > 0)
            def _():
                pltpu.make_async_copy(
                    rows_v.at[b], out_hbm.at[pl.ds(base, _CHUNK)], osem.at[b]
                ).wait()

            gdesc = pltpu.async_copy(
                table_hbm.at[idx_v.at[b]], rows_v.at[b], gsem.at[b]
            )
            gdesc.wait()
            pltpu.async_copy(
                rows_v.at[b], out_hbm.at[pl.ds(base + c * _CHUNK, _CHUNK)], osem.at[b]
            )
            @pl.when(c + _NBUF < _NCHUNK)
            def _():
                idx_start(c + _NBUF, b)

        return carry

    lax.fori_loop(0, _NCHUNK // _NBUF, outer, 0)

    for b in range(_NBUF):
        pltpu.make_async_copy(
            rows_v.at[b], out_hbm.at[pl.ds(base, _CHUNK)], osem.at[b]
        ).wait()


@jax.jit
def _lookup(idx, table):
    f = pl.kernel(
        _lookup_body,
        out_type=jax.ShapeDtypeStruct((_N, _EMBED_DIM), jnp.float32),
        mesh=plsc.VectorSubcoreMesh(core_axis_name="c", subcore_axis_name="s"),
        scratch_types=[
            pltpu.VMEM((_NBUF, _CHUNK), jnp.int32),
            pltpu.VMEM((_NBUF, _CHUNK, _EMBED_DIM), jnp.float32),
            pltpu.SemaphoreType.DMA((_NBUF,)),
            pltpu.SemaphoreType.DMA((_NBUF,)),
            pltpu.SemaphoreType.DMA((_NBUF,)),
        ],
        compiler_params=pltpu.CompilerParams(use_tc_tiling_on_sc=False),
    )
    return f(idx, table)


def kernel(labels, embeddings):
    idx = labels.reshape(-1).astype(jnp.int32)
    out = _lookup(idx, embeddings)
    return out.reshape(_BATCH, _HIST, _EMBED_DIM)
